# Initial kernel scaffold; baseline (speedup 1.0000x reference)
import jax
import jax.numpy as jnp
from jax.experimental import pallas as pl


def kernel(x, W_sw, b_sw, W1, b1, W2, b2):
    raise NotImplementedError("write your pallas kernel here")


# speedup vs baseline: 1.8982x; 1.8982x over previous
import functools

import jax
import jax.numpy as jnp
from jax import lax
from jax.experimental import pallas as pl
from jax.experimental.pallas import tpu as pltpu
from jax.experimental.pallas import tpu_sc as plsc

IN_DIM = 1024
HIDDEN = 2048
N_EXP = 8
TOKENS = 4096
TILE = 128
NTILES = TOKENS // TILE + N_EXP
P = NTILES * TILE
TE_PAD = 64
PM_W = 128

SC_CORES = 2
SC_SUBCORES = 16
NW = SC_CORES * SC_SUBCORES
TPW = TOKENS // NW
CH = 64


def _router_body(xf_ref, wsw_ref, bsw_ref, pos_ref, pmax_ref, te_ref, tv_ref):
    xf = xf_ref[...]
    wsw = wsw_ref[...]
    logits = lax.dot_general(xf, wsw, (((1,), (1,)), ((), ())),
                             preferred_element_type=jnp.float32)
    logits = logits + bsw_ref[...]

    m = jnp.max(logits, axis=1, keepdims=True)
    e = jnp.exp(logits - m)
    s = jnp.sum(e, axis=1, keepdims=True)
    pmax = 1.0 / s

    ids = lax.broadcasted_iota(jnp.int32, (TOKENS, N_EXP), 1)
    routes = jnp.min(jnp.where(logits >= m, ids, N_EXP), axis=1, keepdims=True)
    oh = (ids == routes).astype(jnp.float32)

    c = oh
    sh = 1
    while sh < TOKENS:
        c = c + jnp.concatenate(
            [jnp.zeros((sh, N_EXP), jnp.float32), c[:-sh, :]], axis=0)
        sh *= 2
    rank = jnp.sum(c * oh, axis=1, keepdims=True) - 1.0
    counts = c[TOKENS - 1:TOKENS, :]

    nt = jnp.floor((counts + (TILE - 1)) / TILE)
    tri = (lax.broadcasted_iota(jnp.int32, (N_EXP, N_EXP), 0)
           <= lax.broadcasted_iota(jnp.int32, (N_EXP, N_EXP), 1)
           ).astype(jnp.float32)
    ctiles = lax.dot_general(nt, tri, (((1,), (0,)), ((), ())),
                             preferred_element_type=jnp.float32)
    start_row = (ctiles - nt) * TILE

    pos = jnp.sum(oh * start_row, axis=1, keepdims=True) + rank
    pos_ref[...] = pos.astype(jnp.int32)
    pmax_ref[...] = jnp.broadcast_to(pmax, (TOKENS, PM_W))

    jt = lax.broadcasted_iota(jnp.int32, (TE_PAD, N_EXP), 0)
    ctiles_i = ctiles.astype(jnp.int32)
    te = jnp.sum((jt >= jnp.broadcast_to(ctiles_i, (TE_PAD, N_EXP)))
                 .astype(jnp.int32), axis=1, keepdims=True)
    te_ref[...] = jnp.minimum(te, N_EXP - 1)
    total = jnp.broadcast_to(ctiles_i[0:1, N_EXP - 1:N_EXP], (TE_PAD, 1))
    tv_ref[...] = (jt[:, 0:1] < total).astype(jnp.int32)


_router = pl.pallas_call(
    _router_body,
    out_shape=[
        jax.ShapeDtypeStruct((TOKENS, 1), jnp.int32),
        jax.ShapeDtypeStruct((TOKENS, PM_W), jnp.float32),
        jax.ShapeDtypeStruct((TE_PAD, 1), jnp.int32),
        jax.ShapeDtypeStruct((TE_PAD, 1), jnp.int32),
    ],
)


def _ffn_body(te_ref, tv_ref, xs_ref, w1_ref, b1_ref, w2_ref, b2_ref,
              pm_ref, ys_ref):
    i = pl.program_id(0)

    @pl.when(tv_ref[i] == 1)
    def _():
        x = xs_ref[...]
        h = lax.dot_general(x, w1_ref[0], (((1,), (1,)), ((), ())),
                            preferred_element_type=jnp.float32)
        h = jnp.maximum(h + b1_ref[0], 0.0)
        y = lax.dot_general(h, w2_ref[0], (((1,), (1,)), ((), ())),
                            preferred_element_type=jnp.float32)
        ys_ref[...] = (y + b2_ref[0]) * pm_ref[:, 0:1]


_ffn_grid = pltpu.PrefetchScalarGridSpec(
    num_scalar_prefetch=2,
    grid=(NTILES,),
    in_specs=[
        pl.BlockSpec((TILE, IN_DIM), lambda i, te, tv: (i, 0)),
        pl.BlockSpec((1, HIDDEN, IN_DIM), lambda i, te, tv: (te[i], 0, 0)),
        pl.BlockSpec((1, 1, HIDDEN), lambda i, te, tv: (te[i], 0, 0)),
        pl.BlockSpec((1, IN_DIM, HIDDEN), lambda i, te, tv: (te[i], 0, 0)),
        pl.BlockSpec((1, 1, IN_DIM), lambda i, te, tv: (te[i], 0, 0)),
        pl.BlockSpec((TILE, PM_W), lambda i, te, tv: (i, 0)),
    ],
    out_specs=pl.BlockSpec((TILE, IN_DIM), lambda i, te, tv: (i, 0)),
)

_ffn = pl.pallas_call(
    _ffn_body,
    grid_spec=_ffn_grid,
    out_shape=jax.ShapeDtypeStruct((P, IN_DIM), jnp.float32),
)


def _dispatch_body(xf_hbm, pos_hbm, pmax_hbm, xs_hbm, pms_hbm,
                   idx_v, rows_v, pm_v, sem_a, sem_b):
    wid = lax.axis_index("s") * SC_CORES + lax.axis_index("c")
    for ci in range(TPW // CH):
        base = wid * TPW + ci * CH
        pltpu.sync_copy(pos_hbm.at[pl.ds(base, CH)], idx_v)
        pltpu.sync_copy(xf_hbm.at[pl.ds(base, CH)], rows_v)
        pltpu.sync_copy(pmax_hbm.at[pl.ds(base, CH)], pm_v)
        cp_a = pltpu.async_copy(rows_v, xs_hbm.at[idx_v], sem_a)
        cp_b = pltpu.async_copy(pm_v, pms_hbm.at[idx_v], sem_b)
        cp_a.wait()
        cp_b.wait()


def _combine_body(ys_hbm, pos_hbm, out_hbm, idx_v, rows_v, sem):
    wid = lax.axis_index("s") * SC_CORES + lax.axis_index("c")
    for ci in range(TPW // CH):
        base = wid * TPW + ci * CH
        pltpu.sync_copy(pos_hbm.at[pl.ds(base, CH)], idx_v)
        pltpu.async_copy(ys_hbm.at[idx_v], rows_v, sem).wait()
        pltpu.sync_copy(rows_v, out_hbm.at[pl.ds(base, CH)])


@functools.cache
def _sc_kernels():
    mesh = plsc.VectorSubcoreMesh(core_axis_name="c", subcore_axis_name="s")
    dispatch = functools.partial(
        pl.kernel,
        mesh=mesh,
        out_type=[jax.ShapeDtypeStruct((P, IN_DIM), jnp.float32),
                  jax.ShapeDtypeStruct((P, PM_W), jnp.float32)],
        scratch_types=[pltpu.VMEM((CH,), jnp.int32),
                       pltpu.VMEM((CH, IN_DIM), jnp.float32),
                       pltpu.VMEM((CH, PM_W), jnp.float32),
                       pltpu.SemaphoreType.DMA,
                       pltpu.SemaphoreType.DMA],
    )(_dispatch_body)
    combine = functools.partial(
        pl.kernel,
        mesh=mesh,
        out_type=jax.ShapeDtypeStruct((TOKENS, IN_DIM), jnp.float32),
        scratch_types=[pltpu.VMEM((CH,), jnp.int32),
                       pltpu.VMEM((CH, IN_DIM), jnp.float32),
                       pltpu.SemaphoreType.DMA],
    )(_combine_body)
    return dispatch, combine


def kernel(x, W_sw, b_sw, W1, b1, W2, b2):
    B, N, T, C = x.shape
    xf = x.reshape(-1, C)
    dispatch, combine = _sc_kernels()

    pos2, pmax_rep, te2, tv2 = _router(xf, W_sw, b_sw.reshape(1, N_EXP))
    pos = pos2.reshape(TOKENS)
    te = te2.reshape(TE_PAD)[:NTILES]
    tv = tv2.reshape(TE_PAD)[:NTILES]

    xs, pms = dispatch(xf, pos, pmax_rep)
    ys = _ffn(te, tv, xs, W1, b1.reshape(N_EXP, 1, HIDDEN),
              W2, b2.reshape(N_EXP, 1, IN_DIM), pms)
    out = combine(ys, pos)
    return out.reshape(B, N, T, C)


# speedup vs baseline: 2.5412x; 1.3388x over previous
import functools

import jax
import jax.numpy as jnp
from jax import lax
from jax.experimental import pallas as pl
from jax.experimental.pallas import tpu as pltpu
from jax.experimental.pallas import tpu_sc as plsc

IN_DIM = 1024
HIDDEN = 2048
N_EXP = 8
TOKENS = 4096
TILE = 256
NTILES = TOKENS // TILE + N_EXP
P = NTILES * TILE
TE_PAD = 64
PM_W = 128

SC_CORES = 2
SC_SUBCORES = 16
NW = SC_CORES * SC_SUBCORES
TPW = TOKENS // NW
CH = 64


def _router_body(xf_ref, wsw_ref, bsw_ref, pos_ref, pmax_ref, te_ref, tv_ref):
    xf = xf_ref[...]
    wsw = wsw_ref[...]
    logits = lax.dot_general(xf, wsw, (((1,), (1,)), ((), ())),
                             preferred_element_type=jnp.float32)
    logits = logits + bsw_ref[...]

    m = jnp.max(logits, axis=1, keepdims=True)
    e = jnp.exp(logits - m)
    s = jnp.sum(e, axis=1, keepdims=True)
    pmax = 1.0 / s

    ids = lax.broadcasted_iota(jnp.int32, (TOKENS, N_EXP), 1)
    routes = jnp.min(jnp.where(logits >= m, ids, N_EXP), axis=1, keepdims=True)
    oh = (ids == routes).astype(jnp.float32)

    c = oh
    sh = 1
    while sh < TOKENS:
        c = c + jnp.concatenate(
            [jnp.zeros((sh, N_EXP), jnp.float32), c[:-sh, :]], axis=0)
        sh *= 2
    rank = jnp.sum(c * oh, axis=1, keepdims=True) - 1.0
    counts = c[TOKENS - 1:TOKENS, :]

    nt = jnp.floor((counts + (TILE - 1)) / TILE)
    tri = (lax.broadcasted_iota(jnp.int32, (N_EXP, N_EXP), 0)
           <= lax.broadcasted_iota(jnp.int32, (N_EXP, N_EXP), 1)
           ).astype(jnp.float32)
    ctiles = lax.dot_general(nt, tri, (((1,), (0,)), ((), ())),
                             preferred_element_type=jnp.float32)
    start_row = (ctiles - nt) * TILE

    pos = jnp.sum(oh * start_row, axis=1, keepdims=True) + rank
    pos_ref[...] = pos.astype(jnp.int32)
    pmax_ref[...] = jnp.broadcast_to(pmax, (TOKENS, PM_W))

    jt = lax.broadcasted_iota(jnp.int32, (TE_PAD, N_EXP), 0)
    ctiles_i = ctiles.astype(jnp.int32)
    te = jnp.sum((jt >= jnp.broadcast_to(ctiles_i, (TE_PAD, N_EXP)))
                 .astype(jnp.int32), axis=1, keepdims=True)
    te_ref[...] = jnp.minimum(te, N_EXP - 1)
    total = jnp.broadcast_to(ctiles_i[0:1, N_EXP - 1:N_EXP], (TE_PAD, 1))
    tv_ref[...] = (jt[:, 0:1] < total).astype(jnp.int32)


_router = pl.pallas_call(
    _router_body,
    out_shape=[
        jax.ShapeDtypeStruct((TOKENS, 1), jnp.int32),
        jax.ShapeDtypeStruct((TOKENS, PM_W), jnp.float32),
        jax.ShapeDtypeStruct((TE_PAD, 1), jnp.int32),
        jax.ShapeDtypeStruct((TE_PAD, 1), jnp.int32),
    ],
)


def _ffn_body(te_ref, tv_ref, xs_ref, w1_ref, b1_ref, w2_ref, b2_ref,
              pm_ref, ys_ref):
    i = pl.program_id(0)

    @pl.when(tv_ref[i] == 1)
    def _():
        x = xs_ref[...]
        h = lax.dot_general(x, w1_ref[0], (((1,), (1,)), ((), ())),
                            preferred_element_type=jnp.float32)
        h = jnp.maximum(h + b1_ref[0], 0.0)
        y = lax.dot_general(h, w2_ref[0], (((1,), (1,)), ((), ())),
                            preferred_element_type=jnp.float32)
        ys_ref[...] = (y + b2_ref[0]) * pm_ref[:, 0:1]


_ffn_grid = pltpu.PrefetchScalarGridSpec(
    num_scalar_prefetch=2,
    grid=(NTILES,),
    in_specs=[
        pl.BlockSpec((TILE, IN_DIM), lambda i, te, tv: (i, 0)),
        pl.BlockSpec((1, HIDDEN, IN_DIM), lambda i, te, tv: (te[i], 0, 0)),
        pl.BlockSpec((1, 1, HIDDEN), lambda i, te, tv: (te[i], 0, 0)),
        pl.BlockSpec((1, IN_DIM, HIDDEN), lambda i, te, tv: (te[i], 0, 0)),
        pl.BlockSpec((1, 1, IN_DIM), lambda i, te, tv: (te[i], 0, 0)),
        pl.BlockSpec((TILE, PM_W), lambda i, te, tv: (i, 0)),
    ],
    out_specs=pl.BlockSpec((TILE, IN_DIM), lambda i, te, tv: (i, 0)),
)

_ffn = pl.pallas_call(
    _ffn_body,
    grid_spec=_ffn_grid,
    out_shape=jax.ShapeDtypeStruct((P, IN_DIM), jnp.float32),
)


def _dispatch_body(xf_hbm, pos_hbm, pmax_hbm, xs_hbm, pms_hbm,
                   idx_v, rows_v, pm_v, sem_a, sem_b):
    wid = lax.axis_index("s") * SC_CORES + lax.axis_index("c")
    for ci in range(TPW // CH):
        base = wid * TPW + ci * CH
        pltpu.sync_copy(pos_hbm.at[pl.ds(base, CH)], idx_v)
        pltpu.sync_copy(xf_hbm.at[pl.ds(base, CH)], rows_v)
        pltpu.sync_copy(pmax_hbm.at[pl.ds(base, CH)], pm_v)
        cp_a = pltpu.async_copy(rows_v, xs_hbm.at[idx_v], sem_a)
        cp_b = pltpu.async_copy(pm_v, pms_hbm.at[idx_v], sem_b)
        cp_a.wait()
        cp_b.wait()


def _combine_body(ys_hbm, pos_hbm, out_hbm, idx_v, rows_v, sem):
    wid = lax.axis_index("s") * SC_CORES + lax.axis_index("c")
    for ci in range(TPW // CH):
        base = wid * TPW + ci * CH
        pltpu.sync_copy(pos_hbm.at[pl.ds(base, CH)], idx_v)
        pltpu.async_copy(ys_hbm.at[idx_v], rows_v, sem).wait()
        pltpu.sync_copy(rows_v, out_hbm.at[pl.ds(base, CH)])


@functools.cache
def _sc_kernels():
    mesh = plsc.VectorSubcoreMesh(core_axis_name="c", subcore_axis_name="s")
    dispatch = functools.partial(
        pl.kernel,
        mesh=mesh,
        out_type=[jax.ShapeDtypeStruct((P, IN_DIM), jnp.float32),
                  jax.ShapeDtypeStruct((P, PM_W), jnp.float32)],
        scratch_types=[pltpu.VMEM((CH,), jnp.int32),
                       pltpu.VMEM((CH, IN_DIM), jnp.float32),
                       pltpu.VMEM((CH, PM_W), jnp.float32),
                       pltpu.SemaphoreType.DMA,
                       pltpu.SemaphoreType.DMA],
    )(_dispatch_body)
    combine = functools.partial(
        pl.kernel,
        mesh=mesh,
        out_type=jax.ShapeDtypeStruct((TOKENS, IN_DIM), jnp.float32),
        scratch_types=[pltpu.VMEM((CH,), jnp.int32),
                       pltpu.VMEM((CH, IN_DIM), jnp.float32),
                       pltpu.SemaphoreType.DMA],
    )(_combine_body)
    return dispatch, combine


def kernel(x, W_sw, b_sw, W1, b1, W2, b2):
    B, N, T, C = x.shape
    xf = x.reshape(-1, C)
    dispatch, combine = _sc_kernels()

    pos2, pmax_rep, te2, tv2 = _router(xf, W_sw, b_sw.reshape(1, N_EXP))
    pos = pos2.reshape(TOKENS)
    te = te2.reshape(TE_PAD)[:NTILES]
    tv = tv2.reshape(TE_PAD)[:NTILES]

    xs, pms = dispatch(xf, pos, pmax_rep)
    ys = _ffn(te, tv, xs, W1, b1.reshape(N_EXP, 1, HIDDEN),
              W2, b2.reshape(N_EXP, 1, IN_DIM), pms)
    out = combine(ys, pos)
    return out.reshape(B, N, T, C)


# speedup vs baseline: 2.8247x; 1.1115x over previous
import functools

import jax
import jax.numpy as jnp
from jax import lax
from jax.experimental import pallas as pl
from jax.experimental.pallas import tpu as pltpu
from jax.experimental.pallas import tpu_sc as plsc

IN_DIM = 1024
HIDDEN = 2048
N_EXP = 8
TOKENS = 4096
TILE = 256
NTILES = TOKENS // TILE + N_EXP
P = NTILES * TILE
TE_PAD = 64
PM_W = 128

SC_CORES = 2
SC_SUBCORES = 16
NW = SC_CORES * SC_SUBCORES
TPW = TOKENS // NW
CH = 64


def _router_body(xf_ref, wsw_ref, bsw_ref, pos_ref, pmax_ref, te_ref, tv_ref,
                 wf_ref, bs_ref, se_ref):
    xf = xf_ref[...]
    wsw = wsw_ref[...]
    logits = lax.dot_general(xf, wsw, (((1,), (1,)), ((), ())),
                             preferred_element_type=jnp.float32)
    logits = logits + bsw_ref[...]

    m = jnp.max(logits, axis=1, keepdims=True)
    e = jnp.exp(logits - m)
    s = jnp.sum(e, axis=1, keepdims=True)
    pmax = 1.0 / s

    ids = lax.broadcasted_iota(jnp.int32, (TOKENS, N_EXP), 1)
    routes = jnp.min(jnp.where(logits >= m, ids, N_EXP), axis=1, keepdims=True)
    oh = (ids == routes).astype(jnp.float32)

    c = oh
    sh = 1
    while sh < TOKENS:
        c = c + jnp.concatenate(
            [jnp.zeros((sh, N_EXP), jnp.float32), c[:-sh, :]], axis=0)
        sh *= 2
    rank = jnp.sum(c * oh, axis=1, keepdims=True) - 1.0
    counts = c[TOKENS - 1:TOKENS, :]

    nt = jnp.floor((counts + (TILE - 1)) / TILE)
    tri = (lax.broadcasted_iota(jnp.int32, (N_EXP, N_EXP), 0)
           <= lax.broadcasted_iota(jnp.int32, (N_EXP, N_EXP), 1)
           ).astype(jnp.float32)
    ctiles = lax.dot_general(nt, tri, (((1,), (0,)), ((), ())),
                             preferred_element_type=jnp.float32)
    start_row = (ctiles - nt) * TILE

    pos = jnp.sum(oh * start_row, axis=1, keepdims=True) + rank
    pos_ref[...] = pos.astype(jnp.int32)
    pmax_ref[...] = jnp.broadcast_to(pmax, (TOKENS, PM_W))

    jt = lax.broadcasted_iota(jnp.int32, (TE_PAD, N_EXP), 0)
    ctiles_i = ctiles.astype(jnp.int32)
    te = jnp.sum((jt >= jnp.broadcast_to(ctiles_i, (TE_PAD, N_EXP)))
                 .astype(jnp.int32), axis=1, keepdims=True)
    te = jnp.minimum(te, N_EXP - 1)
    total = jnp.broadcast_to(ctiles_i[0:1, N_EXP - 1:N_EXP], (TE_PAD, 1))
    j1 = jt[:, 0:1]
    tv = (j1 < total).astype(jnp.int32)
    tv_ref[...] = tv

    last_e = jnp.max(jnp.where(tv == 1, te, 0))
    te = jnp.where(tv == 1, te, last_e)
    te_ref[...] = te

    prev_te = jnp.concatenate([te[0:1, :], te[:-1, :]], axis=0)
    wf = jnp.where((te != prev_te) | (j1 == 0), 1, 0)
    rid = wf.astype(jnp.float32)
    sh = 1
    while sh < TE_PAD:
        rid = rid + jnp.concatenate(
            [jnp.zeros((sh, 1), jnp.float32), rid[:-sh, :]], axis=0)
        sh *= 2
    rid = rid.astype(jnp.int32) - 1
    bs = jnp.bitwise_and(rid, 1)
    wf_ref[...] = wf
    bs_ref[...] = bs

    big = jnp.int32(TE_PAD)
    a = jnp.where(wf == 1, j1, big).astype(jnp.float32)
    nb = a
    sh = 1
    while sh < TE_PAD:
        nb = jnp.minimum(nb, jnp.concatenate(
            [nb[sh:, :], jnp.full((sh, 1), float(TE_PAD))], axis=0))
        sh *= 2
    nb = jnp.concatenate([nb[1:, :], jnp.full((1, 1), float(TE_PAD))],
                         axis=0)
    nb_i = nb.astype(jnp.int32)
    oh_nb = (lax.broadcasted_iota(jnp.int32, (TE_PAD, TE_PAD), 1)
             == jnp.broadcast_to(nb_i, (TE_PAD, TE_PAD))).astype(jnp.float32)
    te_at_nb = lax.dot_general(oh_nb, te.astype(jnp.float32),
                               (((1,), (0,)), ((), ())),
                               preferred_element_type=jnp.float32)
    se = jnp.where((wf == 1) & (nb_i < big),
                   te_at_nb.astype(jnp.int32), -1)
    se_ref[...] = se


_router = pl.pallas_call(
    _router_body,
    out_shape=[
        jax.ShapeDtypeStruct((TOKENS, 1), jnp.int32),
        jax.ShapeDtypeStruct((TOKENS, PM_W), jnp.float32),
        jax.ShapeDtypeStruct((TE_PAD, 1), jnp.int32),
        jax.ShapeDtypeStruct((TE_PAD, 1), jnp.int32),
        jax.ShapeDtypeStruct((TE_PAD, 1), jnp.int32),
        jax.ShapeDtypeStruct((TE_PAD, 1), jnp.int32),
        jax.ShapeDtypeStruct((TE_PAD, 1), jnp.int32),
    ],
)


def _ffn_body(te_ref, tv_ref, wf_ref, bs_ref, se_ref,
              xs_ref, w1_ref, b1_ref, w2_ref, b2_ref, pm_ref, ys_ref,
              w1_buf, w2_buf, sem):
    i = pl.program_id(0)
    slot = bs_ref[i]

    @pl.when(i == 0)
    def _():
        pltpu.make_async_copy(w1_ref.at[te_ref[0]], w1_buf.at[0],
                              sem.at[0]).start()
        pltpu.make_async_copy(w2_ref.at[te_ref[0]], w2_buf.at[0],
                              sem.at[0]).start()

    @pl.when(se_ref[i] >= 0)
    def _():
        nxt = se_ref[i]
        pltpu.make_async_copy(w1_ref.at[nxt], w1_buf.at[1 - slot],
                              sem.at[1 - slot]).start()
        pltpu.make_async_copy(w2_ref.at[nxt], w2_buf.at[1 - slot],
                              sem.at[1 - slot]).start()

    @pl.when(wf_ref[i] == 1)
    def _():
        pltpu.make_async_copy(w1_ref.at[0], w1_buf.at[slot],
                              sem.at[slot]).wait()
        pltpu.make_async_copy(w2_ref.at[0], w2_buf.at[slot],
                              sem.at[slot]).wait()

    @pl.when(tv_ref[i] == 1)
    def _():
        x = xs_ref[...]
        h = lax.dot_general(x, w1_buf[slot], (((1,), (1,)), ((), ())),
                            preferred_element_type=jnp.float32)
        h = jnp.maximum(h + b1_ref[0], 0.0)
        y = lax.dot_general(h, w2_buf[slot], (((1,), (1,)), ((), ())),
                            preferred_element_type=jnp.float32)
        ys_ref[...] = (y + b2_ref[0]) * pm_ref[:, 0:1]


_ffn_grid = pltpu.PrefetchScalarGridSpec(
    num_scalar_prefetch=5,
    grid=(NTILES,),
    in_specs=[
        pl.BlockSpec((TILE, IN_DIM), lambda i, *_: (i, 0)),
        pl.BlockSpec(memory_space=pl.ANY),
        pl.BlockSpec((1, 1, HIDDEN), lambda i, te, *_: (te[i], 0, 0)),
        pl.BlockSpec(memory_space=pl.ANY),
        pl.BlockSpec((1, 1, IN_DIM), lambda i, te, *_: (te[i], 0, 0)),
        pl.BlockSpec((TILE, PM_W), lambda i, *_: (i, 0)),
    ],
    out_specs=pl.BlockSpec((TILE, IN_DIM), lambda i, *_: (i, 0)),
    scratch_shapes=[
        pltpu.VMEM((2, HIDDEN, IN_DIM), jnp.float32),
        pltpu.VMEM((2, IN_DIM, HIDDEN), jnp.float32),
        pltpu.SemaphoreType.DMA((2,)),
    ],
)

_ffn = pl.pallas_call(
    _ffn_body,
    grid_spec=_ffn_grid,
    out_shape=jax.ShapeDtypeStruct((P, IN_DIM), jnp.float32),
)


def _dispatch_body(xf_hbm, pos_hbm, pmax_hbm, xs_hbm, pms_hbm,
                   idx_v, rows_v, pm_v, sem_a, sem_b):
    wid = lax.axis_index("s") * SC_CORES + lax.axis_index("c")
    for ci in range(TPW // CH):
        base = wid * TPW + ci * CH
        pltpu.sync_copy(pos_hbm.at[pl.ds(base, CH)], idx_v)
        pltpu.sync_copy(xf_hbm.at[pl.ds(base, CH)], rows_v)
        pltpu.sync_copy(pmax_hbm.at[pl.ds(base, CH)], pm_v)
        cp_a = pltpu.async_copy(rows_v, xs_hbm.at[idx_v], sem_a)
        cp_b = pltpu.async_copy(pm_v, pms_hbm.at[idx_v], sem_b)
        cp_a.wait()
        cp_b.wait()


def _combine_body(ys_hbm, pos_hbm, out_hbm, idx_v, rows_v, sem):
    wid = lax.axis_index("s") * SC_CORES + lax.axis_index("c")
    for ci in range(TPW // CH):
        base = wid * TPW + ci * CH
        pltpu.sync_copy(pos_hbm.at[pl.ds(base, CH)], idx_v)
        pltpu.async_copy(ys_hbm.at[idx_v], rows_v, sem).wait()
        pltpu.sync_copy(rows_v, out_hbm.at[pl.ds(base, CH)])


@functools.cache
def _sc_kernels():
    mesh = plsc.VectorSubcoreMesh(core_axis_name="c", subcore_axis_name="s")
    dispatch = functools.partial(
        pl.kernel,
        mesh=mesh,
        out_type=[jax.ShapeDtypeStruct((P, IN_DIM), jnp.float32),
                  jax.ShapeDtypeStruct((P, PM_W), jnp.float32)],
        scratch_types=[pltpu.VMEM((CH,), jnp.int32),
                       pltpu.VMEM((CH, IN_DIM), jnp.float32),
                       pltpu.VMEM((CH, PM_W), jnp.float32),
                       pltpu.SemaphoreType.DMA,
                       pltpu.SemaphoreType.DMA],
    )(_dispatch_body)
    combine = functools.partial(
        pl.kernel,
        mesh=mesh,
        out_type=jax.ShapeDtypeStruct((TOKENS, IN_DIM), jnp.float32),
        scratch_types=[pltpu.VMEM((CH,), jnp.int32),
                       pltpu.VMEM((CH, IN_DIM), jnp.float32),
                       pltpu.SemaphoreType.DMA],
    )(_combine_body)
    return dispatch, combine


def kernel(x, W_sw, b_sw, W1, b1, W2, b2):
    B, N, T, C = x.shape
    xf = x.reshape(-1, C)
    dispatch, combine = _sc_kernels()

    pos2, pmax_rep, te2, tv2, wf2, bs2, se2 = _router(
        xf, W_sw, b_sw.reshape(1, N_EXP))
    pos = pos2.reshape(TOKENS)
    te = te2.reshape(TE_PAD)[:NTILES]
    tv = tv2.reshape(TE_PAD)[:NTILES]
    wf = wf2.reshape(TE_PAD)[:NTILES]
    bs = bs2.reshape(TE_PAD)[:NTILES]
    se = se2.reshape(TE_PAD)[:NTILES]

    xs, pms = dispatch(xf, pos, pmax_rep)
    ys = _ffn(te, tv, wf, bs, se, xs, W1, b1.reshape(N_EXP, 1, HIDDEN),
              W2, b2.reshape(N_EXP, 1, IN_DIM), pms)
    out = combine(ys, pos)
    return out.reshape(B, N, T, C)


# speedup vs baseline: 2.8507x; 1.0092x over previous
import functools

import jax
import jax.numpy as jnp
from jax import lax
from jax.experimental import pallas as pl
from jax.experimental.pallas import tpu as pltpu
from jax.experimental.pallas import tpu_sc as plsc

IN_DIM = 1024
HIDDEN = 2048
N_EXP = 8
TOKENS = 4096
TILE = 256
NTILES = TOKENS // TILE + N_EXP
P = NTILES * TILE
TE_PAD = 64
PM_W = 128

SC_CORES = 2
SC_SUBCORES = 16
NW = SC_CORES * SC_SUBCORES
TPW = TOKENS // NW
CH = 32
NCH = TPW // CH


def _router_body(xf_ref, wsw_ref, bsw_ref, pos_ref, pmax_ref, te_ref, tv_ref,
                 wf_ref, bs_ref, se_ref):
    xf = xf_ref[...]
    wsw = wsw_ref[...]
    logits = lax.dot_general(xf, wsw, (((1,), (1,)), ((), ())),
                             preferred_element_type=jnp.float32)
    logits = logits + bsw_ref[...]

    m = jnp.max(logits, axis=1, keepdims=True)
    e = jnp.exp(logits - m)
    s = jnp.sum(e, axis=1, keepdims=True)
    pmax = 1.0 / s

    ids = lax.broadcasted_iota(jnp.int32, (TOKENS, N_EXP), 1)
    routes = jnp.min(jnp.where(logits >= m, ids, N_EXP), axis=1, keepdims=True)
    oh = (ids == routes).astype(jnp.float32)

    c = oh
    sh = 1
    while sh < TOKENS:
        c = c + jnp.concatenate(
            [jnp.zeros((sh, N_EXP), jnp.float32), c[:-sh, :]], axis=0)
        sh *= 2
    rank = jnp.sum(c * oh, axis=1, keepdims=True) - 1.0
    counts = c[TOKENS - 1:TOKENS, :]

    nt = jnp.floor((counts + (TILE - 1)) / TILE)
    tri = (lax.broadcasted_iota(jnp.int32, (N_EXP, N_EXP), 0)
           <= lax.broadcasted_iota(jnp.int32, (N_EXP, N_EXP), 1)
           ).astype(jnp.float32)
    ctiles = lax.dot_general(nt, tri, (((1,), (0,)), ((), ())),
                             preferred_element_type=jnp.float32)
    start_row = (ctiles - nt) * TILE

    pos = jnp.sum(oh * start_row, axis=1, keepdims=True) + rank
    pos_ref[...] = pos.astype(jnp.int32)
    pmax_ref[...] = jnp.broadcast_to(pmax, (TOKENS, PM_W))

    jt = lax.broadcasted_iota(jnp.int32, (TE_PAD, N_EXP), 0)
    ctiles_i = ctiles.astype(jnp.int32)
    te = jnp.sum((jt >= jnp.broadcast_to(ctiles_i, (TE_PAD, N_EXP)))
                 .astype(jnp.int32), axis=1, keepdims=True)
    te = jnp.minimum(te, N_EXP - 1)
    total = jnp.broadcast_to(ctiles_i[0:1, N_EXP - 1:N_EXP], (TE_PAD, 1))
    j1 = jt[:, 0:1]
    tv = (j1 < total).astype(jnp.int32)
    tv_ref[...] = tv

    last_e = jnp.max(jnp.where(tv == 1, te, 0))
    te = jnp.where(tv == 1, te, last_e)
    te_ref[...] = te

    prev_te = jnp.concatenate([te[0:1, :], te[:-1, :]], axis=0)
    wf = jnp.where((te != prev_te) | (j1 == 0), 1, 0)
    rid = wf.astype(jnp.float32)
    sh = 1
    while sh < TE_PAD:
        rid = rid + jnp.concatenate(
            [jnp.zeros((sh, 1), jnp.float32), rid[:-sh, :]], axis=0)
        sh *= 2
    rid = rid.astype(jnp.int32) - 1
    bs = jnp.bitwise_and(rid, 1)
    wf_ref[...] = wf
    bs_ref[...] = bs

    big = jnp.int32(TE_PAD)
    a = jnp.where(wf == 1, j1, big).astype(jnp.float32)
    nb = a
    sh = 1
    while sh < TE_PAD:
        nb = jnp.minimum(nb, jnp.concatenate(
            [nb[sh:, :], jnp.full((sh, 1), float(TE_PAD))], axis=0))
        sh *= 2
    nb = jnp.concatenate([nb[1:, :], jnp.full((1, 1), float(TE_PAD))],
                         axis=0)
    nb_i = nb.astype(jnp.int32)
    oh_nb = (lax.broadcasted_iota(jnp.int32, (TE_PAD, TE_PAD), 1)
             == jnp.broadcast_to(nb_i, (TE_PAD, TE_PAD))).astype(jnp.float32)
    te_at_nb = lax.dot_general(oh_nb, te.astype(jnp.float32),
                               (((1,), (0,)), ((), ())),
                               preferred_element_type=jnp.float32)
    se = jnp.where((wf == 1) & (nb_i < big),
                   te_at_nb.astype(jnp.int32), -1)
    se_ref[...] = se


_router = pl.pallas_call(
    _router_body,
    out_shape=[
        jax.ShapeDtypeStruct((TOKENS, 1), jnp.int32),
        jax.ShapeDtypeStruct((TOKENS, PM_W), jnp.float32),
        jax.ShapeDtypeStruct((TE_PAD, 1), jnp.int32),
        jax.ShapeDtypeStruct((TE_PAD, 1), jnp.int32),
        jax.ShapeDtypeStruct((TE_PAD, 1), jnp.int32),
        jax.ShapeDtypeStruct((TE_PAD, 1), jnp.int32),
        jax.ShapeDtypeStruct((TE_PAD, 1), jnp.int32),
    ],
)


def _ffn_body(te_ref, tv_ref, wf_ref, bs_ref, se_ref,
              xs_ref, w1_ref, b1_ref, w2_ref, b2_ref, pm_ref, ys_ref,
              w1_buf, w2_buf, sem):
    i = pl.program_id(0)
    slot = bs_ref[i]

    @pl.when(i == 0)
    def _():
        pltpu.make_async_copy(w1_ref.at[te_ref[0]], w1_buf.at[0],
                              sem.at[0]).start()
        pltpu.make_async_copy(w2_ref.at[te_ref[0]], w2_buf.at[0],
                              sem.at[0]).start()

    @pl.when(se_ref[i] >= 0)
    def _():
        nxt = se_ref[i]
        pltpu.make_async_copy(w1_ref.at[nxt], w1_buf.at[1 - slot],
                              sem.at[1 - slot]).start()
        pltpu.make_async_copy(w2_ref.at[nxt], w2_buf.at[1 - slot],
                              sem.at[1 - slot]).start()

    @pl.when(wf_ref[i] == 1)
    def _():
        pltpu.make_async_copy(w1_ref.at[0], w1_buf.at[slot],
                              sem.at[slot]).wait()
        pltpu.make_async_copy(w2_ref.at[0], w2_buf.at[slot],
                              sem.at[slot]).wait()

    @pl.when(tv_ref[i] == 1)
    def _():
        x = xs_ref[...]
        h = lax.dot_general(x, w1_buf[slot], (((1,), (1,)), ((), ())),
                            preferred_element_type=jnp.float32)
        h = jnp.maximum(h + b1_ref[0], 0.0)
        y = lax.dot_general(h, w2_buf[slot], (((1,), (1,)), ((), ())),
                            preferred_element_type=jnp.float32)
        ys_ref[...] = (y + b2_ref[0]) * pm_ref[:, 0:1]


_ffn_grid = pltpu.PrefetchScalarGridSpec(
    num_scalar_prefetch=5,
    grid=(NTILES,),
    in_specs=[
        pl.BlockSpec((TILE, IN_DIM), lambda i, *_: (i, 0)),
        pl.BlockSpec(memory_space=pl.ANY),
        pl.BlockSpec((1, 1, HIDDEN), lambda i, te, *_: (te[i], 0, 0)),
        pl.BlockSpec(memory_space=pl.ANY),
        pl.BlockSpec((1, 1, IN_DIM), lambda i, te, *_: (te[i], 0, 0)),
        pl.BlockSpec((TILE, PM_W), lambda i, *_: (i, 0)),
    ],
    out_specs=pl.BlockSpec((TILE, IN_DIM), lambda i, *_: (i, 0)),
    scratch_shapes=[
        pltpu.VMEM((2, HIDDEN, IN_DIM), jnp.float32),
        pltpu.VMEM((2, IN_DIM, HIDDEN), jnp.float32),
        pltpu.SemaphoreType.DMA((2,)),
    ],
)

_ffn = pl.pallas_call(
    _ffn_body,
    grid_spec=_ffn_grid,
    out_shape=jax.ShapeDtypeStruct((P, IN_DIM), jnp.float32),
)


def _dispatch_body(xf_hbm, pos_hbm, pmax_hbm, xs_hbm, pms_hbm,
                   idx_v, rows_v, pm_v, sem_i0, sem_i1, sem_o0, sem_o1):
    wid = lax.axis_index("s") * SC_CORES + lax.axis_index("c")
    pltpu.sync_copy(pos_hbm.at[wid], idx_v)
    sem_i = (sem_i0, sem_i1)
    sem_o = (sem_o0, sem_o1)

    def in_cps(ci, base_ci):
        s = ci % 2
        return (pltpu.make_async_copy(xf_hbm.at[pl.ds(base_ci, CH)],
                                      rows_v.at[s], sem_i[s]),
                pltpu.make_async_copy(pmax_hbm.at[pl.ds(base_ci, CH)],
                                      pm_v.at[s], sem_i[s]))

    def out_cps(ci):
        s = ci % 2
        return (pltpu.make_async_copy(rows_v.at[s], xs_hbm.at[idx_v.at[ci]],
                                      sem_o[s]),
                pltpu.make_async_copy(pm_v.at[s], pms_hbm.at[idx_v.at[ci]],
                                      sem_o[s]))

    def start_in(ci):
        for cp in in_cps(ci, wid * TPW + ci * CH):
            cp.start()

    start_in(0)
    start_in(1)
    for ci in range(NCH):
        for cp in in_cps(ci, 0):
            cp.wait()
        for cp in out_cps(ci):
            cp.start()
        if ci + 2 < NCH:
            for cp in out_cps(ci):
                cp.wait()
            start_in(ci + 2)
    for ci in (NCH - 2, NCH - 1):
        for cp in out_cps(ci):
            cp.wait()


def _combine_body(ys_hbm, pos_hbm, out_hbm, idx_v, rows_v,
                  sem_i0, sem_i1, sem_o0, sem_o1):
    wid = lax.axis_index("s") * SC_CORES + lax.axis_index("c")
    pltpu.sync_copy(pos_hbm.at[wid], idx_v)
    sem_i = (sem_i0, sem_i1)
    sem_o = (sem_o0, sem_o1)

    def in_cp(ci):
        s = ci % 2
        return pltpu.make_async_copy(ys_hbm.at[idx_v.at[ci]], rows_v.at[s],
                                     sem_i[s])

    def out_cp(ci, base_ci):
        s = ci % 2
        return pltpu.make_async_copy(rows_v.at[s],
                                     out_hbm.at[pl.ds(base_ci, CH)],
                                     sem_o[s])

    in_cp(0).start()
    in_cp(1).start()
    for ci in range(NCH):
        in_cp(ci).wait()
        out_cp(ci, wid * TPW + ci * CH).start()
        if ci + 2 < NCH:
            out_cp(ci, 0).wait()
            in_cp(ci + 2).start()
    for ci in (NCH - 2, NCH - 1):
        out_cp(ci, 0).wait()


@functools.cache
def _sc_kernels():
    mesh = plsc.VectorSubcoreMesh(core_axis_name="c", subcore_axis_name="s")
    dispatch = functools.partial(
        pl.kernel,
        mesh=mesh,
        out_type=[jax.ShapeDtypeStruct((P, IN_DIM), jnp.float32),
                  jax.ShapeDtypeStruct((P, PM_W), jnp.float32)],
        scratch_types=[pltpu.VMEM((NCH, CH), jnp.int32),
                       pltpu.VMEM((2, CH, IN_DIM), jnp.float32),
                       pltpu.VMEM((2, CH, PM_W), jnp.float32),
                       pltpu.SemaphoreType.DMA,
                       pltpu.SemaphoreType.DMA,
                       pltpu.SemaphoreType.DMA,
                       pltpu.SemaphoreType.DMA],
    )(_dispatch_body)
    combine = functools.partial(
        pl.kernel,
        mesh=mesh,
        out_type=jax.ShapeDtypeStruct((TOKENS, IN_DIM), jnp.float32),
        scratch_types=[pltpu.VMEM((NCH, CH), jnp.int32),
                       pltpu.VMEM((2, CH, IN_DIM), jnp.float32),
                       pltpu.SemaphoreType.DMA,
                       pltpu.SemaphoreType.DMA,
                       pltpu.SemaphoreType.DMA,
                       pltpu.SemaphoreType.DMA],
    )(_combine_body)
    return dispatch, combine


def kernel(x, W_sw, b_sw, W1, b1, W2, b2):
    B, N, T, C = x.shape
    xf = x.reshape(-1, C)
    dispatch, combine = _sc_kernels()

    pos2, pmax_rep, te2, tv2, wf2, bs2, se2 = _router(
        xf, W_sw, b_sw.reshape(1, N_EXP))
    pos = pos2.reshape(NW, NCH, CH)
    te = te2.reshape(TE_PAD)[:NTILES]
    tv = tv2.reshape(TE_PAD)[:NTILES]
    wf = wf2.reshape(TE_PAD)[:NTILES]
    bs = bs2.reshape(TE_PAD)[:NTILES]
    se = se2.reshape(TE_PAD)[:NTILES]

    xs, pms = dispatch(xf, pos, pmax_rep)
    ys = _ffn(te, tv, wf, bs, se, xs, W1, b1.reshape(N_EXP, 1, HIDDEN),
              W2, b2.reshape(N_EXP, 1, IN_DIM), pms)
    out = combine(ys, pos)
    return out.reshape(B, N, T, C)
